# baseline (speedup 1.0000x reference)
import functools

import jax
import jax.numpy as jnp
from jax import lax
from jax.experimental import pallas as pl
from jax.experimental.pallas import tpu as pltpu
from jax.experimental.pallas import tpu_sc as plsc


_BM = 512


def _make_meta_kernel(n, e_num, bm, nt_pad, kk, lanes):
    f32 = jnp.float32
    bf16 = jnp.bfloat16

    def kern(ei_ref, ws_ref, s2_ref, meta_ref, ws_bc_ref):
        ws = ws_ref[...]
        ws_bc_ref[...] = jnp.concatenate(
            [jnp.broadcast_to(ws[:, j:j + 1], (n, lanes)) for j in range(kk)],
            axis=1)

        ei = ei_ref[...].astype(bf16)
        rr = lax.broadcasted_iota(jnp.int32, (n, n), 0)
        cc = lax.broadcasted_iota(jnp.int32, (n, n), 1)
        ident = (rr == cc).astype(bf16)
        tri_lt = (rr < cc).astype(bf16)
        eit = lax.dot_general(ei, ident, (((0,), (0,)), ((), ())),
                              preferred_element_type=f32)
        e0 = eit[0:1, :]
        e1 = eit[1:2, :]
        sub8 = lax.broadcasted_iota(jnp.int32, (e_num, n), 0).astype(f32)
        oh0 = (sub8 == e0).astype(bf16)
        oh1 = (sub8 == e1).astype(bf16)
        c_tok = oh0 + oh1
        cnt_before = jnp.dot(c_tok, tri_lt,
                             preferred_element_type=f32)
        counts = jnp.sum(c_tok.astype(f32), axis=1, keepdims=True)
        tiles_e = jnp.floor((counts + (bm - 1)) * (1.0 / bm))
        pad_e = tiles_e * bm
        r8 = lax.broadcasted_iota(jnp.int32, (e_num, e_num), 0)
        c8 = lax.broadcasted_iota(jnp.int32, (e_num, e_num), 1)
        lt8 = (r8 < c8).astype(f32)
        le8 = (r8 <= c8).astype(f32)
        off_p = lax.dot_general(lt8, pad_e, (((0,), (0,)), ((), ())),
                                preferred_element_type=f32)
        cum_tiles = lax.dot_general(le8, tiles_e, (((0,), (0,)), ((), ())),
                                    preferred_element_type=f32)
        base0 = jnp.sum(oh0.astype(f32) * off_p, axis=0, keepdims=True)
        base1 = jnp.sum(oh1.astype(f32) * off_p, axis=0, keepdims=True)
        within0 = jnp.sum(oh0.astype(f32) * cnt_before, axis=0, keepdims=True)
        within1 = jnp.sum(oh1.astype(f32) * cnt_before, axis=0, keepdims=True)
        same = (e0 == e1).astype(f32)
        s0 = base0 + within0
        s1 = base1 + within1 + same
        s2_ref[...] = jnp.concatenate([s0, s1], axis=0).astype(jnp.int32)

        total = cum_tiles[e_num - 1, 0]
        trow = lax.broadcasted_iota(jnp.int32, (1, nt_pad), 1).astype(f32)
        tc = jnp.minimum(trow, total - 1.0)
        e_t = jnp.sum((cum_tiles <= tc).astype(f32), axis=0, keepdims=True)
        sub8t = lax.broadcasted_iota(jnp.int32, (e_num, nt_pad), 0).astype(f32)
        oh_e = (sub8t == e_t).astype(f32)
        tiles_t = jnp.sum(oh_e * tiles_e, axis=0, keepdims=True)
        cumt_t = jnp.sum(oh_e * cum_tiles, axis=0, keepdims=True)
        offp_t = jnp.sum(oh_e * off_p, axis=0, keepdims=True)
        cnt_t = jnp.sum(oh_e * counts, axis=0, keepdims=True)
        j = tc - (cumt_t - tiles_t)
        blk = offp_t * (1.0 / bm) + j
        hi = jnp.clip(cnt_t - j * bm, 0.0, float(bm))
        hi = jnp.where(trow < total, hi, 0.0)
        meta_ref[...] = jnp.concatenate([blk, e_t, hi], axis=0).astype(
            jnp.int32)

    return kern


def _routing_metadata(expert_indices, expert_weights, e_num, bm, n_tiles,
                      lanes):
    n, k = expert_indices.shape
    nt_pad = 16
    s2, meta, ws_bc = pl.pallas_call(
        _make_meta_kernel(n, e_num, bm, nt_pad, k, lanes),
        out_shape=[
            jax.ShapeDtypeStruct((k, n), jnp.int32),
            jax.ShapeDtypeStruct((3, nt_pad), jnp.int32),
            jax.ShapeDtypeStruct((n, k * lanes), expert_weights.dtype),
        ],
    )(expert_indices.astype(jnp.int32), expert_weights)
    return s2, meta, ws_bc


def _sc_scatter_rows(x, s2, out_rows):
    info = plsc.get_sparse_core_info()
    nc, ns = info.num_cores, info.num_subcores
    nw = nc * ns
    n, d = x.shape
    kk = s2.shape[0]
    t_per_w = n // nw
    mesh = plsc.VectorSubcoreMesh(core_axis_name="c", subcore_axis_name="s")

    @functools.partial(
        pl.kernel, mesh=mesh,
        out_type=jax.ShapeDtypeStruct((out_rows, d), x.dtype),
        scratch_types=[
            pltpu.VMEM((kk, t_per_w), jnp.int32),
            pltpu.VMEM((t_per_w, d), x.dtype),
            pltpu.SemaphoreType.DMA,
        ],
    )
    def k(x_hbm, s2_hbm, out_hbm, idx_v, rows_v, sem):
        wid = lax.axis_index("s") * nc + lax.axis_index("c")
        base = wid * t_per_w
        for j in range(kk):
            pltpu.sync_copy(s2_hbm.at[j, pl.ds(base, t_per_w)], idx_v.at[j])
        pltpu.sync_copy(x_hbm.at[pl.ds(base, t_per_w)], rows_v)
        copies = [
            pltpu.async_copy(rows_v, out_hbm.at[idx_v.at[j]], sem)
            for j in range(kk)
        ]
        for c in copies:
            c.wait()

    return k(x, s2)


def _sc_combine_rows(rows_sorted, s2, ws_bc):
    info = plsc.get_sparse_core_info()
    nc, ns = info.num_cores, info.num_subcores
    nw = nc * ns
    d = rows_sorted.shape[1]
    kk, n_tokens = s2.shape
    t_per_w = n_tokens // nw
    lanes = info.num_lanes
    mesh = plsc.VectorSubcoreMesh(core_axis_name="c", subcore_axis_name="s")

    @functools.partial(
        pl.kernel, mesh=mesh,
        out_type=jax.ShapeDtypeStruct((n_tokens, d), rows_sorted.dtype),
        scratch_types=[
            pltpu.VMEM((kk, t_per_w), jnp.int32),
            pltpu.VMEM((t_per_w, kk * lanes), ws_bc.dtype),
            pltpu.VMEM((kk, t_per_w, d), rows_sorted.dtype),
            pltpu.VMEM((t_per_w, d), rows_sorted.dtype),
            pltpu.SemaphoreType.DMA,
        ],
    )
    def k(rows_hbm, s2_hbm, ws_hbm, out_hbm, idx_v, w_v, rows_v, out_v, sem):
        wid = lax.axis_index("s") * nc + lax.axis_index("c")
        base = wid * t_per_w
        for j in range(kk):
            pltpu.sync_copy(s2_hbm.at[j, pl.ds(base, t_per_w)], idx_v.at[j])
        pltpu.sync_copy(ws_hbm.at[pl.ds(base, t_per_w)], w_v)
        copies = [
            pltpu.async_copy(rows_hbm.at[idx_v.at[j]], rows_v.at[j], sem)
            for j in range(kk)
        ]
        for c in copies:
            c.wait()

        def body(i, carry):
            wvec = [w_v[i, pl.ds(j * lanes, lanes)] for j in range(kk)]
            for c in range(d // lanes):
                sl = pl.ds(c * lanes, lanes)
                acc = rows_v[0, i, sl] * wvec[0]
                for j in range(1, kk):
                    acc = acc + rows_v[j, i, sl] * wvec[j]
                out_v[i, sl] = acc
            return carry

        lax.fori_loop(0, t_per_w, body, 0)
        pltpu.sync_copy(out_v, out_hbm.at[pl.ds(base, t_per_w)])

    return k(rows_sorted, s2, ws_bc)


def _grouped_mm_kernel(meta_ref, xs_ref, wg, wu, wd, out_ref):
    t = pl.program_id(0)
    hi = meta_ref[2, t]

    @pl.when(hi > 0)
    def _compute():
        rows = lax.broadcasted_iota(jnp.int32, (xs_ref.shape[0], 1), 0)
        xb = jnp.where(rows < hi, xs_ref[...], 0.0)
        f32 = jnp.float32
        g = jnp.dot(xb, wg[0], preferred_element_type=f32)
        u = jnp.dot(xb, wu[0], preferred_element_type=f32)
        h = g * lax.logistic(g) * u
        out_ref[...] = jnp.dot(h, wd[0], preferred_element_type=f32)


def _grouped_mm(xs_sorted, w_gate, w_up, w_down, meta, n_tiles):
    nk, d = xs_sorted.shape
    inter = w_gate.shape[2]
    bm = _BM
    grid_spec = pltpu.PrefetchScalarGridSpec(
        num_scalar_prefetch=1,
        grid=(n_tiles,),
        in_specs=[
            pl.BlockSpec((bm, d), lambda t, m: (m[0, t], 0)),
            pl.BlockSpec((1, d, inter), lambda t, m: (m[1, t], 0, 0)),
            pl.BlockSpec((1, d, inter), lambda t, m: (m[1, t], 0, 0)),
            pl.BlockSpec((1, inter, d), lambda t, m: (m[1, t], 0, 0)),
        ],
        out_specs=pl.BlockSpec((bm, d), lambda t, m: (m[0, t], 0)),
    )
    return pl.pallas_call(
        _grouped_mm_kernel,
        grid_spec=grid_spec,
        out_shape=jax.ShapeDtypeStruct((nk, d), xs_sorted.dtype),
        compiler_params=pltpu.CompilerParams(
            dimension_semantics=("arbitrary",),
        ),
    )(meta, xs_sorted, w_gate, w_up, w_down)


def kernel(x, expert_weights, w_gate, w_up, w_down, expert_indices, top_k):
    n, d = x.shape
    e_num = w_gate.shape[0]
    k = expert_indices.shape[1]
    nk = n * k
    bm = _BM
    n_tiles = nk // bm + e_num - 1

    lanes = plsc.get_sparse_core_info().num_lanes
    s2, meta, ws_bc = _routing_metadata(
        expert_indices, expert_weights, e_num, bm, n_tiles, lanes)
    xs_sorted = _sc_scatter_rows(x, s2, n_tiles * bm)
    down = _grouped_mm(xs_sorted, w_gate, w_up, w_down, meta, n_tiles)
    return _sc_combine_rows(down, s2, ws_bc)


# speedup vs baseline: 1.0113x; 1.0113x over previous
import functools

import jax
import jax.numpy as jnp
from jax import lax
from jax.experimental import pallas as pl
from jax.experimental.pallas import tpu as pltpu
from jax.experimental.pallas import tpu_sc as plsc


_BM = 256


def _make_meta_kernel(n, e_num, bm, nt_pad, kk, lanes):
    f32 = jnp.float32
    bf16 = jnp.bfloat16

    def kern(ei_ref, ws_ref, s2_ref, meta_ref, ws_bc_ref):
        ws = ws_ref[...]
        ws_bc_ref[...] = jnp.concatenate(
            [jnp.broadcast_to(ws[:, j:j + 1], (n, lanes)) for j in range(kk)],
            axis=1)

        ei = ei_ref[...].astype(bf16)
        rr = lax.broadcasted_iota(jnp.int32, (n, n), 0)
        cc = lax.broadcasted_iota(jnp.int32, (n, n), 1)
        ident = (rr == cc).astype(bf16)
        tri_lt = (rr < cc).astype(bf16)
        eit = lax.dot_general(ei, ident, (((0,), (0,)), ((), ())),
                              preferred_element_type=f32)
        e0 = eit[0:1, :]
        e1 = eit[1:2, :]
        sub8 = lax.broadcasted_iota(jnp.int32, (e_num, n), 0).astype(f32)
        oh0 = (sub8 == e0).astype(bf16)
        oh1 = (sub8 == e1).astype(bf16)
        c_tok = oh0 + oh1
        cnt_before = jnp.dot(c_tok, tri_lt,
                             preferred_element_type=f32)
        counts = jnp.sum(c_tok.astype(f32), axis=1, keepdims=True)
        tiles_e = jnp.floor((counts + (bm - 1)) * (1.0 / bm))
        pad_e = tiles_e * bm
        r8 = lax.broadcasted_iota(jnp.int32, (e_num, e_num), 0)
        c8 = lax.broadcasted_iota(jnp.int32, (e_num, e_num), 1)
        lt8 = (r8 < c8).astype(f32)
        le8 = (r8 <= c8).astype(f32)
        off_p = lax.dot_general(lt8, pad_e, (((0,), (0,)), ((), ())),
                                preferred_element_type=f32)
        cum_tiles = lax.dot_general(le8, tiles_e, (((0,), (0,)), ((), ())),
                                    preferred_element_type=f32)
        base0 = jnp.sum(oh0.astype(f32) * off_p, axis=0, keepdims=True)
        base1 = jnp.sum(oh1.astype(f32) * off_p, axis=0, keepdims=True)
        within0 = jnp.sum(oh0.astype(f32) * cnt_before, axis=0, keepdims=True)
        within1 = jnp.sum(oh1.astype(f32) * cnt_before, axis=0, keepdims=True)
        same = (e0 == e1).astype(f32)
        s0 = base0 + within0
        s1 = base1 + within1 + same
        s2_ref[...] = jnp.concatenate([s0, s1], axis=0).astype(jnp.int32)

        total = cum_tiles[e_num - 1, 0]
        trow = lax.broadcasted_iota(jnp.int32, (1, nt_pad), 1).astype(f32)
        tc = jnp.minimum(trow, total - 1.0)
        e_t = jnp.sum((cum_tiles <= tc).astype(f32), axis=0, keepdims=True)
        sub8t = lax.broadcasted_iota(jnp.int32, (e_num, nt_pad), 0).astype(f32)
        oh_e = (sub8t == e_t).astype(f32)
        tiles_t = jnp.sum(oh_e * tiles_e, axis=0, keepdims=True)
        cumt_t = jnp.sum(oh_e * cum_tiles, axis=0, keepdims=True)
        offp_t = jnp.sum(oh_e * off_p, axis=0, keepdims=True)
        cnt_t = jnp.sum(oh_e * counts, axis=0, keepdims=True)
        j = tc - (cumt_t - tiles_t)
        blk = offp_t * (1.0 / bm) + j
        hi = jnp.clip(cnt_t - j * bm, 0.0, float(bm))
        hi = jnp.where(trow < total, hi, 0.0)
        meta_ref[...] = jnp.concatenate([blk, e_t, hi], axis=0).astype(
            jnp.int32)

    return kern


def _routing_metadata(expert_indices, expert_weights, e_num, bm, n_tiles,
                      lanes):
    n, k = expert_indices.shape
    nt_pad = 16
    s2, meta, ws_bc = pl.pallas_call(
        _make_meta_kernel(n, e_num, bm, nt_pad, k, lanes),
        out_shape=[
            jax.ShapeDtypeStruct((k, n), jnp.int32),
            jax.ShapeDtypeStruct((3, nt_pad), jnp.int32),
            jax.ShapeDtypeStruct((n, k * lanes), expert_weights.dtype),
        ],
    )(expert_indices.astype(jnp.int32), expert_weights)
    return s2, meta, ws_bc


def _sc_scatter_rows(x, s2, out_rows):
    info = plsc.get_sparse_core_info()
    nc, ns = info.num_cores, info.num_subcores
    nw = nc * ns
    n, d = x.shape
    kk = s2.shape[0]
    t_per_w = n // nw
    mesh = plsc.VectorSubcoreMesh(core_axis_name="c", subcore_axis_name="s")

    @functools.partial(
        pl.kernel, mesh=mesh,
        out_type=jax.ShapeDtypeStruct((out_rows, d), x.dtype),
        scratch_types=[
            pltpu.VMEM((kk, t_per_w), jnp.int32),
            pltpu.VMEM((t_per_w, d), x.dtype),
            pltpu.SemaphoreType.DMA,
        ],
    )
    def k(x_hbm, s2_hbm, out_hbm, idx_v, rows_v, sem):
        wid = lax.axis_index("s") * nc + lax.axis_index("c")
        base = wid * t_per_w
        for j in range(kk):
            pltpu.sync_copy(s2_hbm.at[j, pl.ds(base, t_per_w)], idx_v.at[j])
        pltpu.sync_copy(x_hbm.at[pl.ds(base, t_per_w)], rows_v)
        copies = [
            pltpu.async_copy(rows_v, out_hbm.at[idx_v.at[j]], sem)
            for j in range(kk)
        ]
        for c in copies:
            c.wait()

    return k(x, s2)


def _sc_combine_rows(rows_sorted, s2, ws_bc):
    info = plsc.get_sparse_core_info()
    nc, ns = info.num_cores, info.num_subcores
    nw = nc * ns
    d = rows_sorted.shape[1]
    kk, n_tokens = s2.shape
    t_per_w = n_tokens // nw
    lanes = info.num_lanes
    mesh = plsc.VectorSubcoreMesh(core_axis_name="c", subcore_axis_name="s")

    @functools.partial(
        pl.kernel, mesh=mesh,
        out_type=jax.ShapeDtypeStruct((n_tokens, d), rows_sorted.dtype),
        scratch_types=[
            pltpu.VMEM((kk, t_per_w), jnp.int32),
            pltpu.VMEM((t_per_w, kk * lanes), ws_bc.dtype),
            pltpu.VMEM((kk, t_per_w, d), rows_sorted.dtype),
            pltpu.VMEM((t_per_w, d), rows_sorted.dtype),
            pltpu.SemaphoreType.DMA,
        ],
    )
    def k(rows_hbm, s2_hbm, ws_hbm, out_hbm, idx_v, w_v, rows_v, out_v, sem):
        wid = lax.axis_index("s") * nc + lax.axis_index("c")
        base = wid * t_per_w
        for j in range(kk):
            pltpu.sync_copy(s2_hbm.at[j, pl.ds(base, t_per_w)], idx_v.at[j])
        pltpu.sync_copy(ws_hbm.at[pl.ds(base, t_per_w)], w_v)
        copies = [
            pltpu.async_copy(rows_hbm.at[idx_v.at[j]], rows_v.at[j], sem)
            for j in range(kk)
        ]
        for c in copies:
            c.wait()

        def body(i, carry):
            wvec = [w_v[i, pl.ds(j * lanes, lanes)] for j in range(kk)]
            for c in range(d // lanes):
                sl = pl.ds(c * lanes, lanes)
                acc = rows_v[0, i, sl] * wvec[0]
                for j in range(1, kk):
                    acc = acc + rows_v[j, i, sl] * wvec[j]
                out_v[i, sl] = acc
            return carry

        lax.fori_loop(0, t_per_w, body, 0)
        pltpu.sync_copy(out_v, out_hbm.at[pl.ds(base, t_per_w)])

    return k(rows_sorted, s2, ws_bc)


def _grouped_mm_kernel(meta_ref, xs_ref, wg, wu, wd, out_ref):
    t = pl.program_id(0)
    hi = meta_ref[2, t]

    @pl.when(hi > 0)
    def _compute():
        rows = lax.broadcasted_iota(jnp.int32, (xs_ref.shape[0], 1), 0)
        xb = jnp.where(rows < hi, xs_ref[...], 0.0)
        f32 = jnp.float32
        g = jnp.dot(xb, wg[0], preferred_element_type=f32)
        u = jnp.dot(xb, wu[0], preferred_element_type=f32)
        h = g * lax.logistic(g) * u
        out_ref[...] = jnp.dot(h, wd[0], preferred_element_type=f32)


def _grouped_mm(xs_sorted, w_gate, w_up, w_down, meta, n_tiles):
    nk, d = xs_sorted.shape
    inter = w_gate.shape[2]
    bm = _BM
    grid_spec = pltpu.PrefetchScalarGridSpec(
        num_scalar_prefetch=1,
        grid=(n_tiles,),
        in_specs=[
            pl.BlockSpec((bm, d), lambda t, m: (m[0, t], 0)),
            pl.BlockSpec((1, d, inter), lambda t, m: (m[1, t], 0, 0)),
            pl.BlockSpec((1, d, inter), lambda t, m: (m[1, t], 0, 0)),
            pl.BlockSpec((1, inter, d), lambda t, m: (m[1, t], 0, 0)),
        ],
        out_specs=pl.BlockSpec((bm, d), lambda t, m: (m[0, t], 0)),
    )
    return pl.pallas_call(
        _grouped_mm_kernel,
        grid_spec=grid_spec,
        out_shape=jax.ShapeDtypeStruct((nk, d), xs_sorted.dtype),
        compiler_params=pltpu.CompilerParams(
            dimension_semantics=("arbitrary",),
        ),
    )(meta, xs_sorted, w_gate, w_up, w_down)


def kernel(x, expert_weights, w_gate, w_up, w_down, expert_indices, top_k):
    n, d = x.shape
    e_num = w_gate.shape[0]
    k = expert_indices.shape[1]
    nk = n * k
    bm = _BM
    n_tiles = nk // bm + e_num - 1

    lanes = plsc.get_sparse_core_info().num_lanes
    s2, meta, ws_bc = _routing_metadata(
        expert_indices, expert_weights, e_num, bm, n_tiles, lanes)
    xs_sorted = _sc_scatter_rows(x, s2, n_tiles * bm)
    down = _grouped_mm(xs_sorted, w_gate, w_up, w_down, meta, n_tiles)
    return _sc_combine_rows(down, s2, ws_bc)
